# baseline (speedup 1.0000x reference)
import functools

import jax
import jax.numpy as jnp
from jax import lax
from jax.experimental import pallas as pl
from jax.experimental.pallas import tpu as pltpu
from jax.experimental.pallas import tpu_sc as plsc

N_NODES = 10000
N_EDGES = 320000
DIM = 128
LANES = 16

NC, NS = 2, 16
CHUNK = 128
CPT0 = 53
CPT1 = 105
CPTM = max(CPT0, CPT1)
E0 = NS * CPT0 * CHUNK
E1 = NS * CPT1 * CHUNK
EPAD = E0 + E1
ZROWS = 640
AGG_ROWS = NS * ZROWS
DUMMY_DST = AGG_ROWS - 1
SHIFT = 14


def _sc_segment_sum(x, packed, zrows):
    mesh = plsc.VectorSubcoreMesh(core_axis_name="c", subcore_axis_name="s")

    @functools.partial(
        pl.kernel,
        out_type=jax.ShapeDtypeStruct((NC, AGG_ROWS, DIM), jnp.float32),
        mesh=mesh,
        scratch_types=[
            pltpu.VMEM((CPTM, CHUNK), jnp.int32),
            pltpu.VMEM((2, CHUNK), jnp.int32),
            pltpu.VMEM((2, CHUNK), jnp.int32),
            pltpu.VMEM((CHUNK, DIM), jnp.float32),
            pltpu.VMEM((CHUNK, DIM), jnp.float32),
            pltpu.VMEM_SHARED((AGG_ROWS, DIM), jnp.float32),
            pltpu.SemaphoreType.DMA,
            pltpu.SemaphoreType.DMA,
        ],
    )
    def k(x_hbm, pk_hbm, z_hbm, out_hbm,
          pk_v, src_r, dst_r, rows0, rows1, agg_sh, sem0, sem1):
        c = lax.axis_index("c")
        s = lax.axis_index("s")
        n = lax.select(c == 0, jnp.int32(CPT0), jnp.int32(CPT1))
        pltpu.sync_copy(z_hbm, agg_sh.at[pl.ds(s * ZROWS, ZROWS)])
        pltpu.sync_copy(pk_hbm.at[c, s], pk_v)
        plsc.subcore_barrier()

        def decode_src(j, row):
            for g in range(CHUNK // LANES):
                v = pk_v[j, pl.ds(g * LANES, LANES)]
                src_r[row, pl.ds(g * LANES, LANES)] = lax.bitwise_and(
                    v, (1 << SHIFT) - 1)

        def decode_dst(j, row):
            for g in range(CHUNK // LANES):
                v = pk_v[j, pl.ds(g * LANES, LANES)]
                dst_r[row, pl.ds(g * LANES, LANES)] = lax.shift_right_logical(
                    v, SHIFT)

        rows = (rows0, rows1)
        sems = (sem0, sem1)

        decode_src(0, 0)
        pltpu.async_copy(x_hbm.at[src_r.at[0]], rows0, sem0)

        def step(j, par):
            npar = 1 - par
            decode_src(j + 1, npar)
            pltpu.async_copy(x_hbm.at[src_r.at[npar]], rows[npar], sems[npar])
            decode_dst(j, par)
            pltpu.make_async_copy(
                x_hbm.at[src_r.at[par]], rows[par], sems[par]).wait()
            pltpu.sync_copy(rows[par], agg_sh.at[dst_r.at[par]], add=True)

        @pl.loop(0, (n - 1) // 2)
        def _(i):
            j = i * 2
            step(j, 0)
            step(j + 1, 1)

        decode_dst(n - 1, 0)
        pltpu.make_async_copy(
            x_hbm.at[src_r.at[0]], rows0, sem0).wait()
        pltpu.sync_copy(rows0, agg_sh.at[dst_r.at[0]], add=True)

        plsc.subcore_barrier()
        base = s * ZROWS
        pltpu.sync_copy(agg_sh.at[pl.ds(base, ZROWS)],
                        out_hbm.at[c, pl.ds(base, ZROWS)])

    return k(x, packed, zrows)


def _tc_encoder(x, partials, W, b, prelu_w):
    grid = 10
    blk = N_NODES // grid

    def body(x_ref, p_ref, w_ref, b_ref, pw_ref, hn_ref, hg_ref):
        w = w_ref[...]
        bb = b_ref[...]
        pw = pw_ref[0, 0]

        def enc(m):
            h = lax.dot_general(m, w, (((1,), (1,)), ((), ())),
                                preferred_element_type=jnp.float32,
                                precision=lax.Precision.HIGHEST) + bb
            h = jnp.maximum(h, 0.0) + pw * jnp.minimum(h, 0.0)
            mx = jnp.max(h, axis=1, keepdims=True)
            e = jnp.exp(h - mx)
            return e / jnp.sum(e, axis=1, keepdims=True)

        hn_ref[...] = enc(x_ref[...])
        hg_ref[...] = enc(p_ref[0] + p_ref[1])

    return pl.pallas_call(
        body,
        grid=(grid,),
        in_specs=[
            pl.BlockSpec((blk, DIM), lambda i: (i, 0)),
            pl.BlockSpec((NC, blk, DIM), lambda i: (0, i, 0)),
            pl.BlockSpec((DIM, DIM), lambda i: (0, 0)),
            pl.BlockSpec((1, DIM), lambda i: (0, 0)),
            pl.BlockSpec((1, 1), lambda i: (0, 0)),
        ],
        out_specs=[
            pl.BlockSpec((blk, DIM), lambda i: (i, 0)),
            pl.BlockSpec((blk, DIM), lambda i: (i, 0)),
        ],
        out_shape=[
            jax.ShapeDtypeStruct((N_NODES, DIM), jnp.float32),
            jax.ShapeDtypeStruct((N_NODES, DIM), jnp.float32),
        ],
    )(x, partials, W, b.reshape(1, DIM), prelu_w.reshape(1, 1))


def kernel(x, edge_index, W, b, prelu_w):
    ei = edge_index.astype(jnp.int32)
    pad = EPAD - N_EDGES
    flat = jnp.concatenate(
        [(ei[1] << SHIFT) | ei[0],
         jnp.full((pad,), DUMMY_DST << SHIFT, jnp.int32)])
    packed = jnp.full((NC, NS, CPTM, CHUNK), DUMMY_DST << SHIFT, jnp.int32)
    packed = packed.at[0, :, :CPT0].set(flat[:E0].reshape(NS, CPT0, CHUNK))
    packed = packed.at[1, :, :CPT1].set(flat[E0:].reshape(NS, CPT1, CHUNK))
    zrows = jnp.zeros((ZROWS, DIM), jnp.float32)
    partials = _sc_segment_sum(x, packed, zrows)
    h_node, h_graph = _tc_encoder(x, partials, W, b, prelu_w)
    return (h_node, h_graph)


# speedup vs baseline: 1.0972x; 1.0972x over previous
import functools

import jax
import jax.numpy as jnp
from jax import lax
from jax.experimental import pallas as pl
from jax.experimental.pallas import tpu as pltpu
from jax.experimental.pallas import tpu_sc as plsc

N_NODES = 10000
N_EDGES = 320000
DIM = 128
LANES = 16

NC, NS = 2, 16
CHUNK = 128
CPT0 = 105
CPT1 = 53
CPTM = max(CPT0, CPT1)
E0 = NS * CPT0 * CHUNK
E1 = NS * CPT1 * CHUNK
EPAD = E0 + E1
ZROWS = 640
AGG_ROWS = NS * ZROWS
DUMMY_DST = AGG_ROWS - 1
SHIFT = 14


def _sc_segment_sum(x, packed, zrows):
    mesh = plsc.VectorSubcoreMesh(core_axis_name="c", subcore_axis_name="s")

    @functools.partial(
        pl.kernel,
        out_type=jax.ShapeDtypeStruct((NC, AGG_ROWS, DIM), jnp.float32),
        mesh=mesh,
        scratch_types=[
            pltpu.VMEM((CPTM, CHUNK), jnp.int32),
            pltpu.VMEM((2, CHUNK), jnp.int32),
            pltpu.VMEM((2, CHUNK), jnp.int32),
            pltpu.VMEM((CHUNK, DIM), jnp.float32),
            pltpu.VMEM((CHUNK, DIM), jnp.float32),
            pltpu.VMEM_SHARED((AGG_ROWS, DIM), jnp.float32),
            pltpu.SemaphoreType.DMA,
            pltpu.SemaphoreType.DMA,
        ],
    )
    def k(x_hbm, pk_hbm, z_hbm, out_hbm,
          pk_v, src_r, dst_r, rows0, rows1, agg_sh, sem0, sem1):
        c = lax.axis_index("c")
        s = lax.axis_index("s")
        n = lax.select(c == 0, jnp.int32(CPT0), jnp.int32(CPT1))
        pltpu.sync_copy(z_hbm, agg_sh.at[pl.ds(s * ZROWS, ZROWS)])
        pltpu.sync_copy(pk_hbm.at[c, s], pk_v)
        plsc.subcore_barrier()

        def decode_src(j, row):
            for g in range(CHUNK // LANES):
                v = pk_v[j, pl.ds(g * LANES, LANES)]
                src_r[row, pl.ds(g * LANES, LANES)] = lax.bitwise_and(
                    v, (1 << SHIFT) - 1)

        def decode_dst(j, row):
            for g in range(CHUNK // LANES):
                v = pk_v[j, pl.ds(g * LANES, LANES)]
                dst_r[row, pl.ds(g * LANES, LANES)] = lax.shift_right_logical(
                    v, SHIFT)

        rows = (rows0, rows1)
        sems = (sem0, sem1)

        decode_src(0, 0)
        pltpu.async_copy(x_hbm.at[src_r.at[0]], rows0, sem0)

        def step(j, par):
            npar = 1 - par
            decode_src(j + 1, npar)
            pltpu.async_copy(x_hbm.at[src_r.at[npar]], rows[npar], sems[npar])
            decode_dst(j, par)
            pltpu.make_async_copy(
                x_hbm.at[src_r.at[par]], rows[par], sems[par]).wait()
            pltpu.sync_copy(rows[par], agg_sh.at[dst_r.at[par]], add=True)

        @pl.loop(0, (n - 1) // 2)
        def _(i):
            j = i * 2
            step(j, 0)
            step(j + 1, 1)

        decode_dst(n - 1, 0)
        pltpu.make_async_copy(
            x_hbm.at[src_r.at[0]], rows0, sem0).wait()
        pltpu.sync_copy(rows0, agg_sh.at[dst_r.at[0]], add=True)

        plsc.subcore_barrier()
        base = s * ZROWS
        pltpu.sync_copy(agg_sh.at[pl.ds(base, ZROWS)],
                        out_hbm.at[c, pl.ds(base, ZROWS)])

    return k(x, packed, zrows)


def _tc_encoder(x, partials, W, b, prelu_w):
    grid = 10
    blk = N_NODES // grid

    def body(x_ref, p_ref, w_ref, b_ref, pw_ref, hn_ref, hg_ref):
        w = w_ref[...]
        bb = b_ref[...]
        pw = pw_ref[0, 0]

        def enc(m):
            h = lax.dot_general(m, w, (((1,), (1,)), ((), ())),
                                preferred_element_type=jnp.float32,
                                precision=lax.Precision.HIGHEST) + bb
            h = jnp.maximum(h, 0.0) + pw * jnp.minimum(h, 0.0)
            mx = jnp.max(h, axis=1, keepdims=True)
            e = jnp.exp(h - mx)
            return e / jnp.sum(e, axis=1, keepdims=True)

        hn_ref[...] = enc(x_ref[...])
        hg_ref[...] = enc(p_ref[0] + p_ref[1])

    return pl.pallas_call(
        body,
        grid=(grid,),
        in_specs=[
            pl.BlockSpec((blk, DIM), lambda i: (i, 0)),
            pl.BlockSpec((NC, blk, DIM), lambda i: (0, i, 0)),
            pl.BlockSpec((DIM, DIM), lambda i: (0, 0)),
            pl.BlockSpec((1, DIM), lambda i: (0, 0)),
            pl.BlockSpec((1, 1), lambda i: (0, 0)),
        ],
        out_specs=[
            pl.BlockSpec((blk, DIM), lambda i: (i, 0)),
            pl.BlockSpec((blk, DIM), lambda i: (i, 0)),
        ],
        out_shape=[
            jax.ShapeDtypeStruct((N_NODES, DIM), jnp.float32),
            jax.ShapeDtypeStruct((N_NODES, DIM), jnp.float32),
        ],
    )(x, partials, W, b.reshape(1, DIM), prelu_w.reshape(1, 1))


def kernel(x, edge_index, W, b, prelu_w):
    ei = edge_index.astype(jnp.int32)
    pad = EPAD - N_EDGES
    flat = jnp.concatenate(
        [(ei[1] << SHIFT) | ei[0],
         jnp.full((pad,), DUMMY_DST << SHIFT, jnp.int32)])
    packed = jnp.full((NC, NS, CPTM, CHUNK), DUMMY_DST << SHIFT, jnp.int32)
    packed = packed.at[0, :, :CPT0].set(flat[:E0].reshape(NS, CPT0, CHUNK))
    packed = packed.at[1, :, :CPT1].set(flat[E0:].reshape(NS, CPT1, CHUNK))
    zrows = jnp.zeros((ZROWS, DIM), jnp.float32)
    partials = _sc_segment_sum(x, packed, zrows)
    h_node, h_graph = _tc_encoder(x, partials, W, b, prelu_w)
    return (h_node, h_graph)
